# baseline (speedup 1.0000x reference)
import jax
import jax.numpy as jnp
from jax.experimental import pallas as pl

D_MODEL_ = 32
NUM_EXPERTS_ = 128
N_TOKENS_ = 8192
D_FF_ = 4 * D_MODEL_


def _moe_kernel(x_ref, gw_ref, gb_ref, ew_ref, eb_ref, o_ref):
    xg = x_ref[:NUM_EXPERTS_, :]
    logits = jnp.dot(xg, gw_ref[...].T,
                     preferred_element_type=jnp.float32) + gb_ref[...]
    w = jax.nn.softmax(logits, axis=-1)
    ewt = jnp.transpose(ew_ref[...], (2, 0, 1))
    mt = jnp.sum(ewt * w[None, :, :], axis=1)
    b2 = jnp.sum(w * eb_ref[...], axis=0)
    o_ref[...] = jnp.dot(x_ref[...], mt,
                         preferred_element_type=jnp.float32) + b2[None, :]


def kernel(x, gate_w, gate_b, expert_w, expert_b):
    gb = gate_b.reshape(1, NUM_EXPERTS_)
    return pl.pallas_call(
        _moe_kernel,
        out_shape=jax.ShapeDtypeStruct((N_TOKENS_, NUM_EXPERTS_), jnp.float32),
    )(x, gate_w, gb, expert_w, expert_b)


# speedup vs baseline: 1.3496x; 1.3496x over previous
import jax
import jax.numpy as jnp
from jax.experimental import pallas as pl

D_MODEL_ = 32
NUM_EXPERTS_ = 128
N_TOKENS_ = 8192
D_FF_ = 4 * D_MODEL_


def _moe_kernel(x_ref, gw_ref, gb_ref, ewt_ref, eb_ref, o_ref):
    xg = x_ref[:NUM_EXPERTS_, :]
    logits = jnp.dot(xg, gw_ref[...].T,
                     preferred_element_type=jnp.float32) + gb_ref[...]
    w = jax.nn.softmax(logits, axis=-1)
    mt = jnp.sum(ewt_ref[...] * w[None, :, :], axis=1)
    b2 = jnp.sum(w * eb_ref[...], axis=0)
    o_ref[...] = jnp.dot(x_ref[...].astype(jnp.bfloat16),
                         mt.astype(jnp.bfloat16),
                         preferred_element_type=jnp.float32) + b2[None, :]


def kernel(x, gate_w, gate_b, expert_w, expert_b):
    ewt = jnp.transpose(expert_w, (2, 0, 1))
    gb = gate_b.reshape(1, NUM_EXPERTS_)
    return pl.pallas_call(
        _moe_kernel,
        out_shape=jax.ShapeDtypeStruct((N_TOKENS_, NUM_EXPERTS_), jnp.float32),
    )(x, gate_w, gb, ewt, expert_b)
